# baseline (device time: 85985 ns/iter reference)
import jax
import jax.numpy as jnp
from jax import lax
from jax.experimental import pallas as pl
from jax.experimental.pallas import tpu as pltpu

N_DEV = 32
HR = N_DEV // 2
HL = N_DEV - 1 - HR

RING = [0, 8, 16, 24, 27, 19, 11, 12, 20, 28, 31, 23, 15, 7, 4, 3,
        2, 5, 6, 14, 22, 30, 29, 21, 13, 10, 18, 26, 25, 17, 9, 1]
RIGHT_OF = [0] * N_DEV
LEFT_OF = [0] * N_DEV
for _i, _id in enumerate(RING):
    RIGHT_OF[_id] = RING[(_i + 1) % N_DEV]
    LEFT_OF[_id] = RING[(_i - 1) % N_DEV]


def kernel(q, k, v):
    s_per, d = q.shape
    half = s_per // 2
    scale = 1.0 / (d ** 0.5)

    def body(q_ref, k_ref, v_ref, nbr_ref, out_ref,
             comm_ra, comm_rb, comm_la, comm_lb,
             ss_ra, rs_ra, ss_rb, rs_rb, ss_la, rs_la, ss_lb, rs_lb,
             cap_ra, cap_rb, cap_la, cap_lb):
        left = nbr_ref[0]
        right = nbr_ref[1]

        barrier_sem = pltpu.get_barrier_semaphore()
        for nbr in (left, right):
            pl.semaphore_signal(
                barrier_sem, inc=1,
                device_id=(nbr,), device_id_type=pl.DeviceIdType.MESH,
            )
        pl.semaphore_wait(barrier_sem, 2)

        k16 = k_ref[...].astype(jnp.bfloat16)
        v16 = v_ref[...].astype(jnp.bfloat16)
        for comm in (comm_ra, comm_la):
            comm[0, 0] = k16[:half]
            comm[0, 1] = v16[:half]
        for comm in (comm_rb, comm_lb):
            comm[0, 0] = k16[half:]
            comm[0, 1] = v16[half:]

        q_scaled = (q_ref[...] * scale).astype(jnp.bfloat16)
        dims = (((1,), (1,)), ((), ()))

        l = acc = None

        def update(k_c, v_c):
            nonlocal l, acc
            s = lax.dot_general(q_scaled, k_c, dims,
                                preferred_element_type=jnp.float32)
            p = jnp.exp(s)
            dl = jnp.sum(p, axis=1, keepdims=True)
            da = jnp.dot(p.astype(jnp.bfloat16), v_c,
                         preferred_element_type=jnp.float32)
            l = dl if l is None else l + dl
            acc = da if acc is None else acc + da

        flows = [
            dict(comm=comm_ra, ss=ss_ra, rs=rs_ra, cap=cap_ra,
                 dst=right, csrc=left, H=HR),
            dict(comm=comm_rb, ss=ss_rb, rs=rs_rb, cap=cap_rb,
                 dst=right, csrc=left, H=HR),
            dict(comm=comm_la, ss=ss_la, rs=rs_la, cap=cap_la,
                 dst=left, csrc=right, H=HL),
            dict(comm=comm_lb, ss=ss_lb, rs=rs_lb, cap=cap_lb,
                 dst=left, csrc=right, H=HL),
        ]

        for h in range(HR):
            snd = h % 3
            rcv = (h + 1) % 3
            live = []
            for f in flows:
                if h < f["H"]:
                    if h >= 2:
                        pl.semaphore_wait(f["cap"], 1)
                    rdma = pltpu.make_async_remote_copy(
                        src_ref=f["comm"].at[snd],
                        dst_ref=f["comm"].at[rcv],
                        send_sem=f["ss"].at[snd],
                        recv_sem=f["rs"].at[rcv],
                        device_id=(f["dst"],),
                        device_id_type=pl.DeviceIdType.MESH,
                    )
                    rdma.start()
                    live.append((f, rdma))
            if h == 0:
                update(k16, v16)
            else:
                for f in flows:
                    update(f["comm"][snd, 0], f["comm"][snd, 1])
            for f, rdma in live:
                rdma.wait_send()
                if h <= f["H"] - 3:
                    pl.semaphore_signal(
                        f["cap"], inc=1,
                        device_id=(f["csrc"],),
                        device_id_type=pl.DeviceIdType.MESH,
                    )
            for _, rdma in live:
                rdma.wait_recv()

        update(comm_ra[HR % 3, 0], comm_ra[HR % 3, 1])
        update(comm_rb[HR % 3, 0], comm_rb[HR % 3, 1])
        out_ref[...] = acc / l

    my = lax.axis_index("i")
    nbrs = jnp.stack([
        jnp.array(LEFT_OF, jnp.int32)[my],
        jnp.array(RIGHT_OF, jnp.int32)[my],
    ])

    comm_shape = pltpu.VMEM((3, 2, half, d), jnp.bfloat16)
    return pl.pallas_call(
        body,
        out_shape=jax.ShapeDtypeStruct((s_per, d), jnp.float32),
        in_specs=[pl.BlockSpec(memory_space=pltpu.VMEM)] * 3
        + [pl.BlockSpec(memory_space=pltpu.SMEM)],
        out_specs=pl.BlockSpec(memory_space=pltpu.VMEM),
        scratch_shapes=[comm_shape] * 4 + [
            pltpu.SemaphoreType.DMA((3,)),
            pltpu.SemaphoreType.DMA((3,)),
            pltpu.SemaphoreType.DMA((3,)),
            pltpu.SemaphoreType.DMA((3,)),
            pltpu.SemaphoreType.DMA((3,)),
            pltpu.SemaphoreType.DMA((3,)),
            pltpu.SemaphoreType.DMA((3,)),
            pltpu.SemaphoreType.DMA((3,)),
            pltpu.SemaphoreType.REGULAR,
            pltpu.SemaphoreType.REGULAR,
            pltpu.SemaphoreType.REGULAR,
            pltpu.SemaphoreType.REGULAR,
        ],
        compiler_params=pltpu.CompilerParams(collective_id=0),
    )(q, k, v, nbrs)


# device time: 75956 ns/iter; 1.1320x vs baseline; 1.1320x over previous
import jax
import jax.numpy as jnp
from jax import lax
from jax.experimental import pallas as pl
from jax.experimental.pallas import tpu as pltpu

N_DEV = 32
HR = N_DEV // 2
HL = N_DEV - 1 - HR

RING = [0, 8, 16, 24, 27, 19, 11, 12, 20, 28, 31, 23, 15, 7, 4, 3,
        2, 5, 6, 14, 22, 30, 29, 21, 13, 10, 18, 26, 25, 17, 9, 1]
RIGHT_OF = [0] * N_DEV
LEFT_OF = [0] * N_DEV
for _i, _id in enumerate(RING):
    RIGHT_OF[_id] = RING[(_i + 1) % N_DEV]
    LEFT_OF[_id] = RING[(_i - 1) % N_DEV]


def kernel(q, k, v):
    s_per, d = q.shape
    half = s_per // 2
    scale = 1.0 / (d ** 0.5)

    def body(q_ref, k_ref, v_ref, nbr_ref, out_ref,
             comm_ra, comm_rb, comm_la, comm_lb,
             ss_ra, rs_ra, ss_rb, rs_rb, ss_la, rs_la, ss_lb, rs_lb,
             cap_ra, cap_rb, cap_la, cap_lb):
        left = nbr_ref[0]
        right = nbr_ref[1]

        barrier_sem = pltpu.get_barrier_semaphore()
        for nbr in (left, right):
            pl.semaphore_signal(
                barrier_sem, inc=1,
                device_id=(nbr,), device_id_type=pl.DeviceIdType.MESH,
            )
        pl.semaphore_wait(barrier_sem, 2)

        k16 = k_ref[...].astype(jnp.bfloat16)
        v16 = v_ref[...].astype(jnp.bfloat16)
        for comm in (comm_ra, comm_la):
            comm[0, 0] = k16[:half]
            comm[0, 1] = v16[:half]
        for comm in (comm_rb, comm_lb):
            comm[0, 0] = k16[half:]
            comm[0, 1] = v16[half:]

        q_scaled = (q_ref[...] * scale).astype(jnp.bfloat16)
        dims = (((1,), (1,)), ((), ()))

        l = acc = None

        def update(k_c, v_c):
            nonlocal l, acc
            s = lax.dot_general(q_scaled, k_c, dims,
                                preferred_element_type=jnp.float32)
            p = jnp.exp(s)
            dl = jnp.sum(p, axis=1, keepdims=True)
            da = jnp.dot(p.astype(jnp.bfloat16), v_c,
                         preferred_element_type=jnp.float32)
            l = dl if l is None else l + dl
            acc = da if acc is None else acc + da

        flows = [
            dict(comm=comm_ra, ss=ss_ra, rs=rs_ra, cap=cap_ra,
                 dst=right, csrc=left, H=HR, pend=None),
            dict(comm=comm_rb, ss=ss_rb, rs=rs_rb, cap=cap_rb,
                 dst=right, csrc=left, H=HR, pend=None),
            dict(comm=comm_la, ss=ss_la, rs=rs_la, cap=cap_la,
                 dst=left, csrc=right, H=HL, pend=None),
            dict(comm=comm_lb, ss=ss_lb, rs=rs_lb, cap=cap_lb,
                 dst=left, csrc=right, H=HL, pend=None),
        ]

        for h in range(HR + 1):
            snd = h % 3
            rcv = (h + 1) % 3
            for f in flows:
                if 1 <= h <= f["H"]:
                    f["pend"].wait_recv()
                if h < f["H"]:
                    if h >= 2:
                        pl.semaphore_wait(f["cap"], 1)
                    rdma = pltpu.make_async_remote_copy(
                        src_ref=f["comm"].at[snd],
                        dst_ref=f["comm"].at[rcv],
                        send_sem=f["ss"].at[snd],
                        recv_sem=f["rs"].at[rcv],
                        device_id=(f["dst"],),
                        device_id_type=pl.DeviceIdType.MESH,
                    )
                    rdma.start()
                    f["pend"] = rdma
            if h == 0:
                update(k16, v16)
            else:
                for f in flows:
                    if h - 1 < f["H"]:
                        update(f["comm"][snd, 0], f["comm"][snd, 1])
            for f in flows:
                if h < f["H"]:
                    f["pend"].wait_send()
                    if h <= f["H"] - 3:
                        pl.semaphore_signal(
                            f["cap"], inc=1,
                            device_id=(f["csrc"],),
                            device_id_type=pl.DeviceIdType.MESH,
                        )

        out_ref[...] = acc / l

    my = lax.axis_index("i")
    nbrs = jnp.stack([
        jnp.array(LEFT_OF, jnp.int32)[my],
        jnp.array(RIGHT_OF, jnp.int32)[my],
    ])

    comm_shape = pltpu.VMEM((3, 2, half, d), jnp.bfloat16)
    return pl.pallas_call(
        body,
        out_shape=jax.ShapeDtypeStruct((s_per, d), jnp.float32),
        in_specs=[pl.BlockSpec(memory_space=pltpu.VMEM)] * 3
        + [pl.BlockSpec(memory_space=pltpu.SMEM)],
        out_specs=pl.BlockSpec(memory_space=pltpu.VMEM),
        scratch_shapes=[comm_shape] * 4 + [
            pltpu.SemaphoreType.DMA((3,)),
            pltpu.SemaphoreType.DMA((3,)),
            pltpu.SemaphoreType.DMA((3,)),
            pltpu.SemaphoreType.DMA((3,)),
            pltpu.SemaphoreType.DMA((3,)),
            pltpu.SemaphoreType.DMA((3,)),
            pltpu.SemaphoreType.DMA((3,)),
            pltpu.SemaphoreType.DMA((3,)),
            pltpu.SemaphoreType.REGULAR,
            pltpu.SemaphoreType.REGULAR,
            pltpu.SemaphoreType.REGULAR,
            pltpu.SemaphoreType.REGULAR,
        ],
        compiler_params=pltpu.CompilerParams(collective_id=0),
    )(q, k, v, nbrs)


# device time: 70180 ns/iter; 1.2252x vs baseline; 1.0823x over previous
import jax
import jax.numpy as jnp
from jax import lax
from jax.experimental import pallas as pl
from jax.experimental.pallas import tpu as pltpu

N_DEV = 32
HR = N_DEV // 2
HL = N_DEV - 1 - HR

RING = [0, 8, 16, 24, 27, 19, 11, 12, 20, 28, 31, 23, 15, 7, 4, 3,
        2, 5, 6, 14, 22, 30, 29, 21, 13, 10, 18, 26, 25, 17, 9, 1]
RIGHT_OF = [0] * N_DEV
LEFT_OF = [0] * N_DEV
for _i, _id in enumerate(RING):
    RIGHT_OF[_id] = RING[(_i + 1) % N_DEV]
    LEFT_OF[_id] = RING[(_i - 1) % N_DEV]


def kernel(q, k, v):
    s_per, d = q.shape
    half = s_per // 2
    scale = 1.0 / (d ** 0.5)

    def body(q_ref, k_ref, v_ref, nbr_ref, out_ref,
             comm_ra, comm_rb, comm_la, comm_lb,
             ss_ra, rs_ra, ss_rb, rs_rb, ss_la, rs_la, ss_lb, rs_lb,
             cap_ra, cap_rb, cap_la, cap_lb):
        left = nbr_ref[0]
        right = nbr_ref[1]

        barrier_sem = pltpu.get_barrier_semaphore()
        for nbr in (left, right):
            pl.semaphore_signal(
                barrier_sem, inc=1,
                device_id=(nbr,), device_id_type=pl.DeviceIdType.MESH,
            )
        pl.semaphore_wait(barrier_sem, 2)

        k16 = k_ref[...].astype(jnp.bfloat16)
        v16 = v_ref[...].astype(jnp.bfloat16)
        for comm in (comm_ra, comm_la):
            comm[0, 0] = k16[:half]
            comm[0, 1] = v16[:half]
        for comm in (comm_rb, comm_lb):
            comm[0, 0] = k16[half:]
            comm[0, 1] = v16[half:]

        q_scaled = (q_ref[...] * scale).astype(jnp.bfloat16)
        dims = (((1,), (1,)), ((), ()))

        l = acc = None

        def update(k_c, v_c):
            nonlocal l, acc
            s = lax.dot_general(q_scaled, k_c, dims,
                                preferred_element_type=jnp.float32)
            p = jnp.exp(s)
            dl = jnp.sum(p, axis=1, keepdims=True)
            da = jnp.dot(p.astype(jnp.bfloat16), v_c,
                         preferred_element_type=jnp.float32)
            l = dl if l is None else l + dl
            acc = da if acc is None else acc + da

        flows = [
            dict(comm=comm_ra, ss=ss_ra, rs=rs_ra, cap=cap_ra,
                 dst=right, csrc=left, H=HR, pend=None),
            dict(comm=comm_rb, ss=ss_rb, rs=rs_rb, cap=cap_rb,
                 dst=right, csrc=left, H=HR, pend=None),
            dict(comm=comm_la, ss=ss_la, rs=rs_la, cap=cap_la,
                 dst=left, csrc=right, H=HL, pend=None),
            dict(comm=comm_lb, ss=ss_lb, rs=rs_lb, cap=cap_lb,
                 dst=left, csrc=right, H=HL, pend=None),
        ]

        for h in range(HR + 1):
            snd = h % 3
            rcv = (h + 1) % 3
            for f in flows:
                if 1 <= h <= f["H"]:
                    f["pend"].wait_recv()
                f["sendpend"] = f["pend"]
                if h < f["H"]:
                    if h >= 2:
                        pl.semaphore_wait(f["cap"], 1)
                    rdma = pltpu.make_async_remote_copy(
                        src_ref=f["comm"].at[snd],
                        dst_ref=f["comm"].at[rcv],
                        send_sem=f["ss"].at[snd],
                        recv_sem=f["rs"].at[rcv],
                        device_id=(f["dst"],),
                        device_id_type=pl.DeviceIdType.MESH,
                    )
                    rdma.start()
                    f["pend"] = rdma
            if h == 0:
                update(k16, v16)
            else:
                for f in flows:
                    if h - 1 < f["H"]:
                        update(f["comm"][snd, 0], f["comm"][snd, 1])
            for f in flows:
                if h >= 1 and h - 1 < f["H"]:
                    f["sendpend"].wait_send()
                    if h - 1 <= f["H"] - 3:
                        pl.semaphore_signal(
                            f["cap"], inc=1,
                            device_id=(f["csrc"],),
                            device_id_type=pl.DeviceIdType.MESH,
                        )

        out_ref[...] = acc / l

    my = lax.axis_index("i")
    nbrs = jnp.stack([
        jnp.array(LEFT_OF, jnp.int32)[my],
        jnp.array(RIGHT_OF, jnp.int32)[my],
    ])

    comm_shape = pltpu.VMEM((3, 2, half, d), jnp.bfloat16)
    return pl.pallas_call(
        body,
        out_shape=jax.ShapeDtypeStruct((s_per, d), jnp.float32),
        in_specs=[pl.BlockSpec(memory_space=pltpu.VMEM)] * 3
        + [pl.BlockSpec(memory_space=pltpu.SMEM)],
        out_specs=pl.BlockSpec(memory_space=pltpu.VMEM),
        scratch_shapes=[comm_shape] * 4 + [
            pltpu.SemaphoreType.DMA((3,)),
            pltpu.SemaphoreType.DMA((3,)),
            pltpu.SemaphoreType.DMA((3,)),
            pltpu.SemaphoreType.DMA((3,)),
            pltpu.SemaphoreType.DMA((3,)),
            pltpu.SemaphoreType.DMA((3,)),
            pltpu.SemaphoreType.DMA((3,)),
            pltpu.SemaphoreType.DMA((3,)),
            pltpu.SemaphoreType.REGULAR,
            pltpu.SemaphoreType.REGULAR,
            pltpu.SemaphoreType.REGULAR,
            pltpu.SemaphoreType.REGULAR,
        ],
        compiler_params=pltpu.CompilerParams(collective_id=0),
    )(q, k, v, nbrs)


# device time: 65460 ns/iter; 1.3136x vs baseline; 1.0721x over previous
import jax
import jax.numpy as jnp
from jax import lax
from jax.experimental import pallas as pl
from jax.experimental.pallas import tpu as pltpu

N_DEV = 32
HR = N_DEV // 2
HL = N_DEV - 1 - HR
NSPLIT = 4

RING = [0, 8, 16, 24, 27, 19, 11, 12, 20, 28, 31, 23, 15, 7, 4, 3,
        2, 5, 6, 14, 22, 30, 29, 21, 13, 10, 18, 26, 25, 17, 9, 1]
RIGHT_OF = [0] * N_DEV
LEFT_OF = [0] * N_DEV
for _i, _id in enumerate(RING):
    RIGHT_OF[_id] = RING[(_i + 1) % N_DEV]
    LEFT_OF[_id] = RING[(_i - 1) % N_DEV]


def kernel(q, k, v):
    s_per, d = q.shape
    sub = s_per // NSPLIT
    scale = 1.0 / (d ** 0.5)
    nf = 2 * NSPLIT

    def body(q_ref, k_ref, v_ref, nbr_ref, out_ref, *scratch):
        comms = scratch[:nf]
        ss = scratch[nf:2 * nf]
        rs = scratch[2 * nf:3 * nf]
        caps = scratch[3 * nf:4 * nf]
        left = nbr_ref[0]
        right = nbr_ref[1]

        barrier_sem = pltpu.get_barrier_semaphore()
        for nbr in (left, right):
            pl.semaphore_signal(
                barrier_sem, inc=1,
                device_id=(nbr,), device_id_type=pl.DeviceIdType.MESH,
            )
        pl.semaphore_wait(barrier_sem, 2)

        k16 = k_ref[...].astype(jnp.bfloat16)
        v16 = v_ref[...].astype(jnp.bfloat16)

        flows = []
        for i in range(NSPLIT):
            rows = slice(i * sub, (i + 1) * sub)
            for dst, csrc, H in ((right, left, HR), (left, right, HL)):
                j = len(flows)
                comms[j][0, 0] = k16[rows]
                comms[j][0, 1] = v16[rows]
                flows.append(dict(
                    comm=comms[j], ss=ss[j], rs=rs[j], cap=caps[j],
                    dst=dst, csrc=csrc, H=H, pend=None, sendpend=None,
                ))

        q_scaled = (q_ref[...] * scale).astype(jnp.bfloat16)
        dims = (((1,), (1,)), ((), ()))

        l = acc = None

        def update(k_c, v_c):
            nonlocal l, acc
            s = lax.dot_general(q_scaled, k_c, dims,
                                preferred_element_type=jnp.float32)
            p = jnp.exp(s)
            dl = jnp.sum(p, axis=1, keepdims=True)
            da = jnp.dot(p.astype(jnp.bfloat16), v_c,
                         preferred_element_type=jnp.float32)
            l = dl if l is None else l + dl
            acc = da if acc is None else acc + da

        for h in range(HR + 1):
            snd = h % 3
            rcv = (h + 1) % 3
            for f in flows:
                if 1 <= h <= f["H"]:
                    f["pend"].wait_recv()
                f["sendpend"] = f["pend"]
                if h < f["H"]:
                    if h >= 2:
                        pl.semaphore_wait(f["cap"], 1)
                    rdma = pltpu.make_async_remote_copy(
                        src_ref=f["comm"].at[snd],
                        dst_ref=f["comm"].at[rcv],
                        send_sem=f["ss"].at[snd],
                        recv_sem=f["rs"].at[rcv],
                        device_id=(f["dst"],),
                        device_id_type=pl.DeviceIdType.MESH,
                    )
                    rdma.start()
                    f["pend"] = rdma
            if h == 0:
                update(k16, v16)
            else:
                for f in flows:
                    if h - 1 < f["H"]:
                        update(f["comm"][snd, 0], f["comm"][snd, 1])
            for f in flows:
                if h >= 1 and h - 1 < f["H"]:
                    f["sendpend"].wait_send()
                    if h - 1 <= f["H"] - 3:
                        pl.semaphore_signal(
                            f["cap"], inc=1,
                            device_id=(f["csrc"],),
                            device_id_type=pl.DeviceIdType.MESH,
                        )

        out_ref[...] = acc / l

    my = lax.axis_index("i")
    nbrs = jnp.stack([
        jnp.array(LEFT_OF, jnp.int32)[my],
        jnp.array(RIGHT_OF, jnp.int32)[my],
    ])

    comm_shape = pltpu.VMEM((3, 2, sub, d), jnp.bfloat16)
    return pl.pallas_call(
        body,
        out_shape=jax.ShapeDtypeStruct((s_per, d), jnp.float32),
        in_specs=[pl.BlockSpec(memory_space=pltpu.VMEM)] * 3
        + [pl.BlockSpec(memory_space=pltpu.SMEM)],
        out_specs=pl.BlockSpec(memory_space=pltpu.VMEM),
        scratch_shapes=(
            [comm_shape] * nf
            + [pltpu.SemaphoreType.DMA((3,))] * nf
            + [pltpu.SemaphoreType.DMA((3,))] * nf
            + [pltpu.SemaphoreType.REGULAR] * nf
        ),
        compiler_params=pltpu.CompilerParams(collective_id=0),
    )(q, k, v, nbrs)
